# baseline (speedup 1.0000x reference)
import functools

import jax
import jax.numpy as jnp
from jax import lax
from jax.experimental import pallas as pl
from jax.experimental.pallas import tpu as pltpu
from jax.experimental.pallas import tpu_sc as plsc

BATCH = 16384
EMBED_DIM = 64
LANES = 16
IDX_CHUNK = 128


def _mf_body(users_ref, items_ref, ue_ref, ie_ref, ub_ref, ib_ref, gb_ref,
             out_ref, uidx_v, iidx_v, urows_v, irows_v, ub_v, ib_v,
             scores_v, gb_v, sem, *, b_per_w, num_cores):
    wid = lax.axis_index("s") * num_cores + lax.axis_index("c")
    base = wid * b_per_w

    pltpu.sync_copy(users_ref.at[pl.ds(base, b_per_w)], uidx_v)
    pltpu.sync_copy(items_ref.at[pl.ds(base, b_per_w)], iidx_v)
    pltpu.sync_copy(gb_ref, gb_v.at[pl.ds(0, 1)])

    handles = []
    for j in range(b_per_w // IDX_CHUNK):
        sl = pl.ds(j * IDX_CHUNK, IDX_CHUNK)
        handles.append(pltpu.async_copy(
            ue_ref.at[uidx_v.at[sl]], urows_v.at[sl], sem))
        handles.append(pltpu.async_copy(
            ie_ref.at[iidx_v.at[sl]], irows_v.at[sl], sem))
        handles.append(pltpu.async_copy(
            ub_ref.at[uidx_v.at[sl]], ub_v.at[sl], sem))
        handles.append(pltpu.async_copy(
            ib_ref.at[iidx_v.at[sl]], ib_v.at[sl], sem))
    for h in handles:
        h.wait()

    gb = gb_v[pl.ds(0, LANES)][0]
    lane = lax.iota(jnp.int32, LANES)

    def group(g, _):
        r0 = g * LANES
        rows = r0 + lane
        acc = jnp.zeros((LANES,), jnp.float32)
        for c in range(EMBED_DIM):
            col = jnp.full((LANES,), c, jnp.int32)
            u = plsc.load_gather(urows_v, [rows, col])
            v = plsc.load_gather(irows_v, [rows, col])
            acc = acc + u * v
        acc = acc + ub_v[pl.ds(r0, LANES)] + ib_v[pl.ds(r0, LANES)] + gb
        scores_v[pl.ds(r0, LANES)] = acc
        return 0

    lax.fori_loop(0, b_per_w // LANES, group, 0)

    pltpu.sync_copy(scores_v, out_ref.at[pl.ds(base, b_per_w)])


def kernel(users, items, user_embedding, item_embedding, user_bias,
           item_bias, global_bias):
    info = plsc.get_sparse_core_info()
    num_workers = info.num_cores * info.num_subcores
    b_per_w = BATCH // num_workers

    mesh = plsc.VectorSubcoreMesh(core_axis_name="c", subcore_axis_name="s")
    k = pl.kernel(
        functools.partial(_mf_body, b_per_w=b_per_w,
                          num_cores=info.num_cores),
        mesh=mesh,
        compiler_params=pltpu.CompilerParams(needs_layout_passes=False,
                                             use_tc_tiling_on_sc=False),
        out_type=jax.ShapeDtypeStruct((BATCH,), jnp.float32),
        scratch_types=[
            pltpu.VMEM((b_per_w,), jnp.int32),
            pltpu.VMEM((b_per_w,), jnp.int32),
            pltpu.VMEM((b_per_w, EMBED_DIM), jnp.float32),
            pltpu.VMEM((b_per_w, EMBED_DIM), jnp.float32),
            pltpu.VMEM((b_per_w,), jnp.float32),
            pltpu.VMEM((b_per_w,), jnp.float32),
            pltpu.VMEM((b_per_w,), jnp.float32),
            pltpu.VMEM((LANES,), jnp.float32),
            pltpu.SemaphoreType.DMA,
        ],
    )
    return k(users.astype(jnp.int32), items.astype(jnp.int32),
             user_embedding, item_embedding,
             user_bias.reshape(user_bias.shape[0]),
             item_bias.reshape(item_bias.shape[0]),
             global_bias)
